# baseline (speedup 1.0000x reference)
import functools

import jax
import jax.numpy as jnp
from jax import lax
from jax.experimental import pallas as pl
from jax.experimental.pallas import tpu as pltpu
from jax.experimental.pallas import tpu_sc as plsc

N = 10000
E = 320000
H = 128
DE = 16
NC, NS = 2, 16
NW = NC * NS
K = 80
CPW = (E // NW) // K
ROWS = E // K

_mesh = plsc.VectorSubcoreMesh(core_axis_name="c", subcore_axis_name="s")


@functools.partial(
    pl.kernel, mesh=_mesh,
    out_type=jax.ShapeDtypeStruct((NC, N), jnp.float32),
    scratch_types=[
        pltpu.VMEM((CPW, K), jnp.int32),
        pltpu.VMEM((CPW, K), jnp.float32),
        pltpu.VMEM_SHARED((N,), jnp.float32),
    ],
)
def _sc_deg(dst_hbm, ew_hbm, zer_hbm, out_hbm, didx, ewv, acc):
    c = lax.axis_index("c")
    s = lax.axis_index("s")
    w = c * NS + s
    pltpu.sync_copy(dst_hbm.at[w], didx)
    pltpu.sync_copy(ew_hbm.at[w], ewv)

    @pl.when(s == 0)
    def _():
        pltpu.sync_copy(zer_hbm, acc)

    plsc.subcore_barrier()

    def chunk(j, carry):
        pltpu.sync_copy(ewv.at[j], acc.at[didx.at[j]], add=True)
        return carry

    lax.fori_loop(0, CPW, chunk, 0)
    plsc.subcore_barrier()

    @pl.when(s == 0)
    def _():
        pltpu.sync_copy(acc, out_hbm.at[c])


@functools.partial(
    pl.kernel, mesh=_mesh,
    out_type=jax.ShapeDtypeStruct((NC, N, H), jnp.float32),
    scratch_types=[
        pltpu.VMEM((8, K), jnp.int32),
        pltpu.VMEM((2, 1, K), jnp.float32),
        pltpu.VMEM((2, K, H), jnp.float32),
        pltpu.VMEM_SHARED((N, H), jnp.float32),
        pltpu.SemaphoreType.DMA,
        pltpu.SemaphoreType.DMA,
        pltpu.SemaphoreType.DMA,
        pltpu.SemaphoreType.DMA,
        pltpu.SemaphoreType.DMA,
        pltpu.SemaphoreType.DMA,
    ],
)
def _sc_layer(hp_hbm, sde_hbm, ew_hbm, zer_hbm, out_hbm,
              idx4, ewb, rows, acc, gsem0, gsem1, isem0, isem1, isem2, isem3):
    c = lax.axis_index("c")
    s = lax.axis_index("s")
    w = c * NS + s
    gsems = (gsem0, gsem1)
    isems = (isem0, isem1, isem2, isem3)

    @pl.when(s == 0)
    def _():
        pltpu.sync_copy(zer_hbm, acc)

    plsc.subcore_barrier()

    def issue_idx(cur, r):
        pltpu.async_copy(sde_hbm.at[w, cur], idx4.at[pl.ds(2 * r, 2)],
                         isems[r])

    def wait_idx(r):
        pltpu.make_async_copy(sde_hbm.at[w, 0], idx4.at[pl.ds(2 * r, 2)],
                              isems[r]).wait()

    def issue_gather(cur, r, b):
        pltpu.async_copy(hp_hbm.at[idx4.at[2 * r]], rows.at[b], gsems[b])
        pltpu.async_copy(ew_hbm.at[w, cur], ewb.at[b], gsems[b])

    def wait_gather(r, b):
        pltpu.make_async_copy(hp_hbm.at[idx4.at[2 * r]], rows.at[b],
                              gsems[b]).wait()
        pltpu.make_async_copy(ew_hbm.at[w, 0], ewb.at[b], gsems[b]).wait()

    for t in range(4):
        issue_idx(t, t)
    wait_idx(0)
    wait_idx(1)
    issue_gather(0, 0, 0)
    issue_gather(1, 1, 1)

    def process(cur, t):
        b = t % 2
        r2 = (t + 2) % 4
        wait_gather(t, b)

        def scale16(g, c2):
            wv = ewb[b, 0, pl.ds(g * 16, 16)]
            for l in range(16):
                wgt = wv[l]
                i = g * 16 + l
                for q in range(H // 16):
                    sl = pl.ds(q * 16, 16)
                    rows[b, i, sl] = rows[b, i, sl] * wgt
            return c2

        lax.fori_loop(0, K // 16, scale16, 0, unroll=True)
        pltpu.sync_copy(rows.at[b], acc.at[idx4.at[2 * t + 1]], add=True)

        def prefetch_idx():
            issue_idx(cur + 4, t)

        def next_gather():
            wait_idx(r2)
            issue_gather(cur + 2, r2, b)

        if isinstance(cur, int):
            if cur + 4 < CPW:
                prefetch_idx()
            if cur + 2 < CPW:
                next_gather()
        else:
            @pl.when(cur + 4 < CPW)
            def _():
                prefetch_idx()

            @pl.when(cur + 2 < CPW)
            def _():
                next_gather()

    @pl.loop(0, CPW - 1, step=4)
    def _(j):
        for t in range(4):
            process(j + t, t)

    process(CPW - 1, 0)
    plsc.subcore_barrier()

    @pl.when(s == 0)
    def _():
        pltpu.sync_copy(acc, out_hbm.at[c])


@functools.partial(
    pl.kernel, mesh=_mesh,
    out_type=jax.ShapeDtypeStruct((ROWS, K, H), jnp.float32),
    scratch_types=[
        pltpu.VMEM((CPW, K), jnp.int32),
        pltpu.VMEM((CPW, K), jnp.int32),
        pltpu.VMEM((2, K, H), jnp.float32),
        pltpu.VMEM((2, K, H), jnp.float32),
        pltpu.VMEM((2, K, H), jnp.float32),
        pltpu.SemaphoreType.DMA,
        pltpu.SemaphoreType.DMA,
        pltpu.SemaphoreType.DMA,
        pltpu.SemaphoreType.DMA,
    ],
)
def _sc_decoder(hs_hbm, hd_hbm, src_hbm, dst_hbm, out_hbm,
                sidx_all, didx_all, ra, rb, ob, gsem0, gsem1, osem0, osem1):
    c = lax.axis_index("c")
    s = lax.axis_index("s")
    w = c * NS + s
    base = w * CPW
    pltpu.sync_copy(src_hbm.at[w], sidx_all)
    pltpu.sync_copy(dst_hbm.at[w], didx_all)
    gsems = (gsem0, gsem1)
    osems = (osem0, osem1)

    def issue(cur, b):
        pltpu.async_copy(hs_hbm.at[sidx_all.at[cur]], ra.at[b], gsems[b])
        pltpu.async_copy(hd_hbm.at[didx_all.at[cur]], rb.at[b], gsems[b])

    def wait_gather(cur, b):
        pltpu.make_async_copy(hs_hbm.at[sidx_all.at[cur]], ra.at[b],
                              gsems[b]).wait()
        pltpu.make_async_copy(hd_hbm.at[didx_all.at[cur]], rb.at[b],
                              gsems[b]).wait()

    def wait_store(cur, b):
        pltpu.make_async_copy(ob.at[b], out_hbm.at[base + cur],
                              osems[b]).wait()

    issue(0, 0)
    issue(1, 1)

    def process(cur, b):
        if isinstance(cur, int):
            if cur >= 2:
                wait_store(cur - 2, b)
        else:
            @pl.when(cur >= 2)
            def _():
                wait_store(cur - 2, b)
        wait_gather(cur, b)

        def addrow(i, c2):
            for q in range(H // 16):
                sl = pl.ds(q * 16, 16)
                ob[b, i, sl] = ra[b, i, sl] + rb[b, i, sl]
            return c2

        lax.fori_loop(0, K, addrow, 0, unroll=8)
        pltpu.async_copy(ob.at[b], out_hbm.at[base + cur], osems[b])
        if isinstance(cur, int):
            if cur + 2 < CPW:
                issue(cur + 2, b)
        else:
            @pl.when(cur + 2 < CPW)
            def _():
                issue(cur + 2, b)

    @pl.loop(0, CPW - 1, step=2)
    def _(j):
        process(j, 0)
        process(j + 1, 1)

    process(CPW - 1, 0)
    wait_store(CPW - 2, 1)
    wait_store(CPW - 1, 0)


def _tc_prep_body(x_ref, wx_ref, bx_ref, dpt_ref, h1p_ref, dinv_ref):
    dp = dpt_ref[...]
    deg = dp[:, 0:1] + dp[:, 1:2]
    dinv = jnp.where(deg > 0, lax.rsqrt(jnp.maximum(deg, 1e-12)), 0.0)
    h = jnp.dot(x_ref[...], wx_ref[...],
                preferred_element_type=jnp.float32) + bx_ref[...]
    h1p_ref[...] = h * dinv
    dinv_ref[...] = dinv


def _tc_mid_body(p0_ref, p1_ref, dinv_ref, w_ref, b_ref, out_ref):
    dinv = dinv_ref[...]
    agg = (p0_ref[...] + p1_ref[...]) * dinv
    h = jnp.dot(agg, w_ref[...], preferred_element_type=jnp.float32) + b_ref[...]
    out_ref[...] = jnp.maximum(h, 0.0) * dinv


def _tc_post_body(p0_ref, p1_ref, dinv_ref, w_ref, b_ref, wd1_ref, we_ref,
                  be_ref, bd1_ref, hs_ref, hd_ref, cmat_ref):
    agg = (p0_ref[...] + p1_ref[...]) * dinv_ref[...]
    h = jnp.dot(agg, w_ref[...], preferred_element_type=jnp.float32) + b_ref[...]
    h = jnp.maximum(h, 0.0)
    wd1 = wd1_ref[...]
    a = wd1[0:H, :]
    b2 = wd1[H:2 * H, :]
    c0 = wd1[2 * H:3 * H, :]
    cmat = jnp.dot(we_ref[...], c0, preferred_element_type=jnp.float32)
    bconst = bd1_ref[...] + jnp.dot(be_ref[...], c0,
                                    preferred_element_type=jnp.float32)
    hs_ref[...] = jnp.dot(h, a, preferred_element_type=jnp.float32)
    hd_ref[...] = jnp.dot(h, b2, preferred_element_type=jnp.float32) + bconst
    cmat_ref[...] = cmat


def _tc_final_body(g_ref, ea_ref, cmat_ref, wd2_ref, bd2_ref, out_ref):
    v = g_ref[...] + jnp.dot(ea_ref[...], cmat_ref[...],
                             preferred_element_type=jnp.float32)
    v = jnp.maximum(v, 0.0)
    out_ref[...] = jnp.dot(v, wd2_ref[...],
                           preferred_element_type=jnp.float32) + bd2_ref[...]


_EB = 2000


def kernel(x, edge_index, edge_attr, edge_weight,
           Wx, bx, We, be, Wg0, bg0, Wg1, bg1, Wd1, bd1, Wd2, bd2):
    f32 = jnp.float32
    src2 = edge_index[0].reshape(NW, CPW, K)
    dst2 = edge_index[1].reshape(NW, CPW, K)
    ew2 = edge_weight.reshape(NW, CPW, K)
    zer_n = jnp.zeros((N,), f32)
    zer_nh = jnp.zeros((N, H), f32)

    sde = jnp.stack([src2, dst2], axis=2)
    ew4 = ew2.reshape(NW, CPW, 1, K)

    deg_parts = _sc_deg(dst2, ew2, zer_n)
    dpt = deg_parts.T

    h1p, dinv = pl.pallas_call(
        _tc_prep_body,
        out_shape=[jax.ShapeDtypeStruct((N, H), f32),
                   jax.ShapeDtypeStruct((N, 1), f32)],
    )(x, Wx, bx.reshape(1, H), dpt)

    parts1 = _sc_layer(h1p, sde, ew4, zer_nh)
    h2p = pl.pallas_call(
        _tc_mid_body,
        out_shape=jax.ShapeDtypeStruct((N, H), f32),
    )(parts1[0], parts1[1], dinv, Wg0, bg0.reshape(1, H))

    parts2 = _sc_layer(h2p, sde, ew4, zer_nh)
    hs, hd, cmat = pl.pallas_call(
        _tc_post_body,
        out_shape=[jax.ShapeDtypeStruct((N, H), f32),
                   jax.ShapeDtypeStruct((N, H), f32),
                   jax.ShapeDtypeStruct((DE, H), f32)],
    )(parts2[0], parts2[1], dinv, Wg1, bg1.reshape(1, H), Wd1,
      We, be.reshape(1, H), bd1.reshape(1, H))

    g = _sc_decoder(hs, hd, src2, dst2)
    g2 = g.reshape(E, H)

    out = pl.pallas_call(
        _tc_final_body,
        grid=(E // _EB,),
        in_specs=[
            pl.BlockSpec((_EB, H), lambda i: (i, 0)),
            pl.BlockSpec((_EB, DE), lambda i: (i, 0)),
            pl.BlockSpec((DE, H), lambda i: (0, 0)),
            pl.BlockSpec((H, 1), lambda i: (0, 0)),
            pl.BlockSpec((1, 1), lambda i: (0, 0)),
        ],
        out_specs=pl.BlockSpec((_EB, 1), lambda i: (i, 0)),
        out_shape=jax.ShapeDtypeStruct((E, 1), f32),
    )(g2, edge_attr, cmat, Wd2, bd2.reshape(1, 1))
    return out


# speedup vs baseline: 1.5917x; 1.5917x over previous
import functools

import jax
import jax.numpy as jnp
from jax import lax
from jax.experimental import pallas as pl
from jax.experimental.pallas import tpu as pltpu
from jax.experimental.pallas import tpu_sc as plsc

N = 10000
E = 320000
H = 128
DE = 16
NC, NS = 2, 16
NW = NC * NS
K = 80
CPW = (E // NW) // K
ROWS = E // K

_mesh = plsc.VectorSubcoreMesh(core_axis_name="c", subcore_axis_name="s")


@functools.partial(
    pl.kernel, mesh=_mesh,
    out_type=jax.ShapeDtypeStruct((NC, N), jnp.float32),
    scratch_types=[
        pltpu.VMEM((CPW, K), jnp.int32),
        pltpu.VMEM((CPW, K), jnp.float32),
        pltpu.VMEM_SHARED((N,), jnp.float32),
    ],
)
def _sc_deg(dst_hbm, ew_hbm, zer_hbm, out_hbm, didx, ewv, acc):
    c = lax.axis_index("c")
    s = lax.axis_index("s")
    w = c * NS + s
    pltpu.sync_copy(dst_hbm.at[w], didx)
    pltpu.sync_copy(ew_hbm.at[w], ewv)

    @pl.when(s == 0)
    def _():
        pltpu.sync_copy(zer_hbm, acc)

    plsc.subcore_barrier()

    def chunk(j, carry):
        pltpu.sync_copy(ewv.at[j], acc.at[didx.at[j]], add=True)
        return carry

    lax.fori_loop(0, CPW, chunk, 0)
    plsc.subcore_barrier()

    @pl.when(s == 0)
    def _():
        pltpu.sync_copy(acc, out_hbm.at[c])


@functools.partial(
    pl.kernel, mesh=_mesh,
    out_type=jax.ShapeDtypeStruct((NC, N, H), jnp.float32),
    scratch_types=[
        pltpu.VMEM((8, K), jnp.int32),
        pltpu.VMEM((2, 1, K), jnp.float32),
        pltpu.VMEM((2, K, H), jnp.float32),
        pltpu.VMEM_SHARED((N, H), jnp.float32),
        pltpu.SemaphoreType.DMA,
        pltpu.SemaphoreType.DMA,
        pltpu.SemaphoreType.DMA,
        pltpu.SemaphoreType.DMA,
        pltpu.SemaphoreType.DMA,
        pltpu.SemaphoreType.DMA,
    ],
)
def _sc_layer(hp_hbm, sde_hbm, ew_hbm, zer_hbm, out_hbm,
              idx4, ewb, rows, acc, gsem0, gsem1, isem0, isem1, isem2, isem3):
    c = lax.axis_index("c")
    s = lax.axis_index("s")
    w = c * NS + s
    gsems = (gsem0, gsem1)
    isems = (isem0, isem1, isem2, isem3)

    @pl.when(s == 0)
    def _():
        pltpu.sync_copy(zer_hbm, acc)

    plsc.subcore_barrier()

    def issue_idx(cur, r):
        pltpu.async_copy(sde_hbm.at[w, cur], idx4.at[pl.ds(2 * r, 2)],
                         isems[r])

    def wait_idx(r):
        pltpu.make_async_copy(sde_hbm.at[w, 0], idx4.at[pl.ds(2 * r, 2)],
                              isems[r]).wait()

    def issue_gather(cur, r, b):
        pltpu.async_copy(hp_hbm.at[idx4.at[2 * r]], rows.at[b], gsems[b])
        pltpu.async_copy(ew_hbm.at[w, cur], ewb.at[b], gsems[b])

    def wait_gather(r, b):
        pltpu.make_async_copy(hp_hbm.at[idx4.at[2 * r]], rows.at[b],
                              gsems[b]).wait()
        pltpu.make_async_copy(ew_hbm.at[w, 0], ewb.at[b], gsems[b]).wait()

    for t in range(4):
        issue_idx(t, t)
    wait_idx(0)
    wait_idx(1)
    issue_gather(0, 0, 0)
    issue_gather(1, 1, 1)

    def process(cur, t):
        b = t % 2
        r2 = (t + 2) % 4
        wait_gather(t, b)

        def scale16(g, c2):
            wv = ewb[b, 0, pl.ds(g * 16, 16)]
            for l in range(16):
                wgt = wv[l]
                i = g * 16 + l
                for q in range(H // 16):
                    sl = pl.ds(q * 16, 16)
                    rows[b, i, sl] = rows[b, i, sl] * wgt
            return c2

        lax.fori_loop(0, K // 16, scale16, 0)
        pltpu.sync_copy(rows.at[b], acc.at[idx4.at[2 * t + 1]], add=True)

        def prefetch_idx():
            issue_idx(cur + 4, t)

        def next_gather():
            wait_idx(r2)
            issue_gather(cur + 2, r2, b)

        if isinstance(cur, int):
            if cur + 4 < CPW:
                prefetch_idx()
            if cur + 2 < CPW:
                next_gather()
        else:
            @pl.when(cur + 4 < CPW)
            def _():
                prefetch_idx()

            @pl.when(cur + 2 < CPW)
            def _():
                next_gather()

    @pl.loop(0, CPW - 1, step=4)
    def _(j):
        for t in range(4):
            process(j + t, t)

    process(CPW - 1, 0)
    plsc.subcore_barrier()

    @pl.when(s == 0)
    def _():
        pltpu.sync_copy(acc, out_hbm.at[c])


@functools.partial(
    pl.kernel, mesh=_mesh,
    out_type=jax.ShapeDtypeStruct((E, H), jnp.float32),
    scratch_types=[
        pltpu.VMEM((CPW, K), jnp.int32),
        pltpu.VMEM((CPW, K), jnp.int32),
        pltpu.VMEM((2, K, H), jnp.float32),
        pltpu.VMEM((2, K, H), jnp.float32),
        pltpu.VMEM((2, K, H), jnp.float32),
        pltpu.SemaphoreType.DMA,
        pltpu.SemaphoreType.DMA,
        pltpu.SemaphoreType.DMA,
        pltpu.SemaphoreType.DMA,
    ],
)
def _sc_decoder(hs_hbm, hd_hbm, src_hbm, dst_hbm, out_hbm,
                sidx_all, didx_all, ra, rb, ob, gsem0, gsem1, osem0, osem1):
    c = lax.axis_index("c")
    s = lax.axis_index("s")
    w = c * NS + s
    base = w * CPW
    pltpu.sync_copy(src_hbm.at[w], sidx_all)
    pltpu.sync_copy(dst_hbm.at[w], didx_all)
    gsems = (gsem0, gsem1)
    osems = (osem0, osem1)

    def issue(cur, b):
        pltpu.async_copy(hs_hbm.at[sidx_all.at[cur]], ra.at[b], gsems[b])
        pltpu.async_copy(hd_hbm.at[didx_all.at[cur]], rb.at[b], gsems[b])

    def wait_gather(cur, b):
        pltpu.make_async_copy(hs_hbm.at[sidx_all.at[cur]], ra.at[b],
                              gsems[b]).wait()
        pltpu.make_async_copy(hd_hbm.at[didx_all.at[cur]], rb.at[b],
                              gsems[b]).wait()

    def wait_store(cur, b):
        pltpu.make_async_copy(ob.at[b], out_hbm.at[pl.ds((base + cur) * K, K)],
                              osems[b]).wait()

    issue(0, 0)
    issue(1, 1)

    def process(cur, b):
        if isinstance(cur, int):
            if cur >= 2:
                wait_store(cur - 2, b)
        else:
            @pl.when(cur >= 2)
            def _():
                wait_store(cur - 2, b)
        wait_gather(cur, b)

        def addrow(i, c2):
            for q in range(H // 16):
                sl = pl.ds(q * 16, 16)
                ob[b, i, sl] = ra[b, i, sl] + rb[b, i, sl]
            return c2

        lax.fori_loop(0, K, addrow, 0)
        pltpu.async_copy(ob.at[b], out_hbm.at[pl.ds((base + cur) * K, K)],
                         osems[b])
        if isinstance(cur, int):
            if cur + 2 < CPW:
                issue(cur + 2, b)
        else:
            @pl.when(cur + 2 < CPW)
            def _():
                issue(cur + 2, b)

    @pl.loop(0, CPW - 1, step=2)
    def _(j):
        process(j, 0)
        process(j + 1, 1)

    process(CPW - 1, 0)
    wait_store(CPW - 2, 1)
    wait_store(CPW - 1, 0)


def _tc_prep_body(x_ref, wx_ref, bx_ref, dpt_ref, h1p_ref, dinv_ref):
    dp = dpt_ref[...]
    deg = dp[:, 0:1] + dp[:, 1:2]
    dinv = jnp.where(deg > 0, lax.rsqrt(jnp.maximum(deg, 1e-12)), 0.0)
    h = jnp.dot(x_ref[...], wx_ref[...],
                preferred_element_type=jnp.float32) + bx_ref[...]
    h1p_ref[...] = h * dinv
    dinv_ref[...] = dinv


def _tc_mid_body(p0_ref, p1_ref, dinv_ref, w_ref, b_ref, out_ref):
    dinv = dinv_ref[...]
    agg = (p0_ref[...] + p1_ref[...]) * dinv
    h = jnp.dot(agg, w_ref[...], preferred_element_type=jnp.float32) + b_ref[...]
    out_ref[...] = jnp.maximum(h, 0.0) * dinv


def _tc_post_body(p0_ref, p1_ref, dinv_ref, w_ref, b_ref, wd1_ref, we_ref,
                  be_ref, bd1_ref, hs_ref, hd_ref, cmat_ref):
    agg = (p0_ref[...] + p1_ref[...]) * dinv_ref[...]
    h = jnp.dot(agg, w_ref[...], preferred_element_type=jnp.float32) + b_ref[...]
    h = jnp.maximum(h, 0.0)
    wd1 = wd1_ref[...]
    a = wd1[0:H, :]
    b2 = wd1[H:2 * H, :]
    c0 = wd1[2 * H:3 * H, :]
    cmat = jnp.dot(we_ref[...], c0, preferred_element_type=jnp.float32)
    bconst = bd1_ref[...] + jnp.dot(be_ref[...], c0,
                                    preferred_element_type=jnp.float32)
    hs_ref[...] = jnp.dot(h, a, preferred_element_type=jnp.float32)
    hd_ref[...] = jnp.dot(h, b2, preferred_element_type=jnp.float32) + bconst
    cmat_ref[...] = cmat


def _tc_final_body(g_ref, ea_ref, cmat_ref, wd2_ref, bd2_ref, out_ref):
    v = g_ref[...] + jnp.dot(ea_ref[...], cmat_ref[...],
                             preferred_element_type=jnp.float32)
    v = jnp.maximum(v, 0.0)
    v3 = v.reshape(_EB // H, H, H)
    r = lax.dot_general(v3, wd2_ref[...], (((2,), (0,)), ((), ())),
                        preferred_element_type=jnp.float32)
    out_ref[...] = (r[:, :, 0] + bd2_ref[...])[None]


_EB = 3200


def kernel(x, edge_index, edge_attr, edge_weight,
           Wx, bx, We, be, Wg0, bg0, Wg1, bg1, Wd1, bd1, Wd2, bd2):
    f32 = jnp.float32
    src2 = edge_index[0].reshape(NW, CPW, K)
    dst2 = edge_index[1].reshape(NW, CPW, K)
    ew2 = edge_weight.reshape(NW, CPW, K)
    zer_n = jnp.zeros((N,), f32)
    zer_nh = jnp.zeros((N, H), f32)

    sde = jnp.stack([src2, dst2], axis=2)
    ew4 = ew2.reshape(NW, CPW, 1, K)

    deg_parts = _sc_deg(dst2, ew2, zer_n)
    dpt = deg_parts.T

    h1p, dinv = pl.pallas_call(
        _tc_prep_body,
        out_shape=[jax.ShapeDtypeStruct((N, H), f32),
                   jax.ShapeDtypeStruct((N, 1), f32)],
    )(x, Wx, bx.reshape(1, H), dpt)

    parts1 = _sc_layer(h1p, sde, ew4, zer_nh)
    h2p = pl.pallas_call(
        _tc_mid_body,
        out_shape=jax.ShapeDtypeStruct((N, H), f32),
    )(parts1[0], parts1[1], dinv, Wg0, bg0.reshape(1, H))

    parts2 = _sc_layer(h2p, sde, ew4, zer_nh)
    hs, hd, cmat = pl.pallas_call(
        _tc_post_body,
        out_shape=[jax.ShapeDtypeStruct((N, H), f32),
                   jax.ShapeDtypeStruct((N, H), f32),
                   jax.ShapeDtypeStruct((DE, H), f32)],
    )(parts2[0], parts2[1], dinv, Wg1, bg1.reshape(1, H), Wd1,
      We, be.reshape(1, H), bd1.reshape(1, H))

    g2 = _sc_decoder(hs, hd, src2, dst2)

    out = pl.pallas_call(
        _tc_final_body,
        grid=(E // _EB,),
        in_specs=[
            pl.BlockSpec((_EB, H), lambda i: (i, 0)),
            pl.BlockSpec((_EB, DE), lambda i: (i, 0)),
            pl.BlockSpec((DE, H), lambda i: (0, 0)),
            pl.BlockSpec((H, 1), lambda i: (0, 0)),
            pl.BlockSpec((1, 1), lambda i: (0, 0)),
        ],
        out_specs=pl.BlockSpec((1, _EB // H, H), lambda i: (i, 0, 0)),
        out_shape=jax.ShapeDtypeStruct((E // _EB, _EB // H, H), f32),
    )(g2, edge_attr, cmat, Wd2, bd2.reshape(1, 1))
    return out.reshape(E, 1)


# speedup vs baseline: 1.7275x; 1.0853x over previous
import functools

import jax
import jax.numpy as jnp
from jax import lax
from jax.experimental import pallas as pl
from jax.experimental.pallas import tpu as pltpu
from jax.experimental.pallas import tpu_sc as plsc

N = 10000
E = 320000
H = 128
DE = 16
NC, NS = 2, 16
NW = NC * NS
K = 80
CPW = (E // NW) // K
ROWS = E // K

_mesh = plsc.VectorSubcoreMesh(core_axis_name="c", subcore_axis_name="s")


@functools.partial(
    pl.kernel, mesh=_mesh,
    out_type=jax.ShapeDtypeStruct((NC, N), jnp.float32),
    scratch_types=[
        pltpu.VMEM((CPW, K), jnp.int32),
        pltpu.VMEM((CPW, K), jnp.float32),
        pltpu.VMEM_SHARED((N,), jnp.float32),
    ],
)
def _sc_deg(dst_hbm, ew_hbm, zer_hbm, out_hbm, didx, ewv, acc):
    c = lax.axis_index("c")
    s = lax.axis_index("s")
    w = c * NS + s
    pltpu.sync_copy(dst_hbm.at[w], didx)
    pltpu.sync_copy(ew_hbm.at[w], ewv)

    @pl.when(s == 0)
    def _():
        pltpu.sync_copy(zer_hbm, acc)

    plsc.subcore_barrier()

    def chunk(j, carry):
        pltpu.sync_copy(ewv.at[j], acc.at[didx.at[j]], add=True)
        return carry

    lax.fori_loop(0, CPW, chunk, 0)
    plsc.subcore_barrier()

    @pl.when(s == 0)
    def _():
        pltpu.sync_copy(acc, out_hbm.at[c])


@functools.partial(
    pl.kernel, mesh=_mesh,
    out_type=jax.ShapeDtypeStruct((NC, N, H), jnp.float32),
    scratch_types=[
        pltpu.VMEM((8, K), jnp.int32),
        pltpu.VMEM((2, 1, K), jnp.float32),
        pltpu.VMEM((2, K, H), jnp.float32),
        pltpu.VMEM_SHARED((N, H), jnp.float32),
        pltpu.SemaphoreType.DMA,
        pltpu.SemaphoreType.DMA,
        pltpu.SemaphoreType.DMA,
        pltpu.SemaphoreType.DMA,
        pltpu.SemaphoreType.DMA,
        pltpu.SemaphoreType.DMA,
    ],
)
def _sc_layer(hp_hbm, sde_hbm, ew_hbm, zer_hbm, out_hbm,
              idx4, ewb, rows, acc, gsem0, gsem1, isem0, isem1, isem2, isem3):
    c = lax.axis_index("c")
    s = lax.axis_index("s")
    w = c * NS + s
    gsems = (gsem0, gsem1)
    isems = (isem0, isem1, isem2, isem3)

    @pl.when(s == 0)
    def _():
        pltpu.sync_copy(zer_hbm, acc)

    plsc.subcore_barrier()

    def issue_idx(cur, r):
        pltpu.async_copy(sde_hbm.at[w, cur], idx4.at[pl.ds(2 * r, 2)],
                         isems[r])

    def wait_idx(r):
        pltpu.make_async_copy(sde_hbm.at[w, 0], idx4.at[pl.ds(2 * r, 2)],
                              isems[r]).wait()

    def issue_gather(cur, r, b):
        pltpu.async_copy(hp_hbm.at[idx4.at[2 * r]], rows.at[b], gsems[b])
        pltpu.async_copy(ew_hbm.at[w, cur], ewb.at[b], gsems[b])

    def wait_gather(r, b):
        pltpu.make_async_copy(hp_hbm.at[idx4.at[2 * r]], rows.at[b],
                              gsems[b]).wait()
        pltpu.make_async_copy(ew_hbm.at[w, 0], ewb.at[b], gsems[b]).wait()

    for t in range(4):
        issue_idx(t, t)
    wait_idx(0)
    wait_idx(1)
    issue_gather(0, 0, 0)
    issue_gather(1, 1, 1)

    def process(cur, t):
        b = t % 2
        r2 = (t + 2) % 4
        wait_gather(t, b)

        def scale16(g, c2):
            wv = ewb[b, 0, pl.ds(g * 16, 16)]
            for l in range(16):
                wgt = wv[l]
                i = g * 16 + l
                for q in range(H // 16):
                    sl = pl.ds(q * 16, 16)
                    rows[b, i, sl] = rows[b, i, sl] * wgt
            return c2

        lax.fori_loop(0, K // 16, scale16, 0)
        pltpu.sync_copy(rows.at[b], acc.at[idx4.at[2 * t + 1]], add=True)

        def prefetch_idx():
            issue_idx(cur + 4, t)

        def next_gather():
            wait_idx(r2)
            issue_gather(cur + 2, r2, b)

        if isinstance(cur, int):
            if cur + 4 < CPW:
                prefetch_idx()
            if cur + 2 < CPW:
                next_gather()
        else:
            @pl.when(cur + 4 < CPW)
            def _():
                prefetch_idx()

            @pl.when(cur + 2 < CPW)
            def _():
                next_gather()

    @pl.loop(0, CPW - 1, step=4)
    def _(j):
        for t in range(4):
            process(j + t, t)

    process(CPW - 1, 0)
    plsc.subcore_barrier()

    @pl.when(s == 0)
    def _():
        pltpu.sync_copy(acc, out_hbm.at[c])


@functools.partial(
    pl.kernel, mesh=_mesh,
    out_type=jax.ShapeDtypeStruct((E, H), jnp.float32),
    scratch_types=[
        pltpu.VMEM((CPW, K), jnp.int32),
        pltpu.VMEM((CPW, K), jnp.int32),
        pltpu.VMEM((2, K, H), jnp.float32),
        pltpu.VMEM((2, K, H), jnp.float32),
        pltpu.VMEM((2, K, H), jnp.float32),
        pltpu.SemaphoreType.DMA,
        pltpu.SemaphoreType.DMA,
        pltpu.SemaphoreType.DMA,
        pltpu.SemaphoreType.DMA,
    ],
)
def _sc_decoder(hs_hbm, hd_hbm, src_hbm, dst_hbm, out_hbm,
                sidx_all, didx_all, ra, rb, ob, gsem0, gsem1, osem0, osem1):
    c = lax.axis_index("c")
    s = lax.axis_index("s")
    w = c * NS + s
    base = w * CPW
    pltpu.sync_copy(src_hbm.at[w], sidx_all)
    pltpu.sync_copy(dst_hbm.at[w], didx_all)
    gsems = (gsem0, gsem1)
    osems = (osem0, osem1)

    def issue(cur, b):
        pltpu.async_copy(hs_hbm.at[sidx_all.at[cur]], ra.at[b], gsems[b])
        pltpu.async_copy(hd_hbm.at[didx_all.at[cur]], rb.at[b], gsems[b])

    def wait_gather(cur, b):
        pltpu.make_async_copy(hs_hbm.at[sidx_all.at[cur]], ra.at[b],
                              gsems[b]).wait()
        pltpu.make_async_copy(hd_hbm.at[didx_all.at[cur]], rb.at[b],
                              gsems[b]).wait()

    def wait_store(cur, b):
        pltpu.make_async_copy(ob.at[b], out_hbm.at[pl.ds((base + cur) * K, K)],
                              osems[b]).wait()

    issue(0, 0)
    issue(1, 1)

    def process(cur, b):
        if isinstance(cur, int):
            if cur >= 2:
                wait_store(cur - 2, b)
        else:
            @pl.when(cur >= 2)
            def _():
                wait_store(cur - 2, b)
        wait_gather(cur, b)

        def addrow(i, c2):
            for q in range(H // 16):
                sl = pl.ds(q * 16, 16)
                ob[b, i, sl] = ra[b, i, sl] + rb[b, i, sl]
            return c2

        lax.fori_loop(0, K, addrow, 0)
        pltpu.async_copy(ob.at[b], out_hbm.at[pl.ds((base + cur) * K, K)],
                         osems[b])
        if isinstance(cur, int):
            if cur + 2 < CPW:
                issue(cur + 2, b)
        else:
            @pl.when(cur + 2 < CPW)
            def _():
                issue(cur + 2, b)

    @pl.loop(0, CPW - 1, step=2)
    def _(j):
        process(j, 0)
        process(j + 1, 1)

    process(CPW - 1, 0)
    wait_store(CPW - 2, 1)
    wait_store(CPW - 1, 0)


def _tc_prep_body(x_ref, wx_ref, bx_ref, dpt_ref, h1p_ref, dinv_ref):
    dp = dpt_ref[...]
    deg = dp[:, 0:1] + dp[:, 1:2]
    dinv = jnp.where(deg > 0, lax.rsqrt(jnp.maximum(deg, 1e-12)), 0.0)
    h = jnp.dot(x_ref[...], wx_ref[...],
                preferred_element_type=jnp.float32) + bx_ref[...]
    h1p_ref[...] = h * dinv
    dinv_ref[...] = dinv


def _tc_mid_body(p0_ref, p1_ref, dinv_ref, w_ref, b_ref, out_ref):
    dinv = dinv_ref[...]
    agg = (p0_ref[...] + p1_ref[...]) * dinv
    h = jnp.dot(agg, w_ref[...], preferred_element_type=jnp.float32) + b_ref[...]
    out_ref[...] = jnp.maximum(h, 0.0) * dinv


def _tc_post_body(p0_ref, p1_ref, dinv_ref, w_ref, b_ref, wd1_ref, we_ref,
                  be_ref, bd1_ref, hs_ref, hd_ref, cmat_ref):
    agg = (p0_ref[...] + p1_ref[...]) * dinv_ref[...]
    h = jnp.dot(agg, w_ref[...], preferred_element_type=jnp.float32) + b_ref[...]
    h = jnp.maximum(h, 0.0)
    wd1 = wd1_ref[...]
    a = wd1[0:H, :]
    b2 = wd1[H:2 * H, :]
    c0 = wd1[2 * H:3 * H, :]
    cmat = jnp.dot(we_ref[...], c0, preferred_element_type=jnp.float32)
    bconst = bd1_ref[...] + jnp.dot(be_ref[...], c0,
                                    preferred_element_type=jnp.float32)
    hs_ref[...] = jnp.dot(h, a, preferred_element_type=jnp.float32)
    hd_ref[...] = jnp.dot(h, b2, preferred_element_type=jnp.float32) + bconst
    cmat_ref[...] = cmat


def _tc_final_body(g_ref, eat_ref, cmat_ref, wd2_ref, bd2_ref, out_ref):
    v = g_ref[...] + lax.dot_general(eat_ref[...], cmat_ref[...],
                                     (((0,), (0,)), ((), ())),
                                     preferred_element_type=jnp.float32)
    v = jnp.maximum(v, 0.0)
    v3 = v.reshape(_EB // H, H, H)
    r = lax.dot_general(v3, wd2_ref[...], (((2,), (0,)), ((), ())),
                        preferred_element_type=jnp.float32)
    out_ref[...] = (r[:, :, 0] + bd2_ref[...])[None]


_EB = 3200


def kernel(x, edge_index, edge_attr, edge_weight,
           Wx, bx, We, be, Wg0, bg0, Wg1, bg1, Wd1, bd1, Wd2, bd2):
    f32 = jnp.float32
    src2 = edge_index[0].reshape(NW, CPW, K)
    dst2 = edge_index[1].reshape(NW, CPW, K)
    ew2 = edge_weight.reshape(NW, CPW, K)
    zer_n = jnp.zeros((N,), f32)
    zer_nh = jnp.zeros((N, H), f32)

    sde = jnp.stack([src2, dst2], axis=2)
    ew4 = ew2.reshape(NW, CPW, 1, K)

    deg_parts = _sc_deg(dst2, ew2, zer_n)
    dpt = deg_parts.T

    h1p, dinv = pl.pallas_call(
        _tc_prep_body,
        out_shape=[jax.ShapeDtypeStruct((N, H), f32),
                   jax.ShapeDtypeStruct((N, 1), f32)],
    )(x, Wx, bx.reshape(1, H), dpt)

    parts1 = _sc_layer(h1p, sde, ew4, zer_nh)
    h2p = pl.pallas_call(
        _tc_mid_body,
        out_shape=jax.ShapeDtypeStruct((N, H), f32),
    )(parts1[0], parts1[1], dinv, Wg0, bg0.reshape(1, H))

    parts2 = _sc_layer(h2p, sde, ew4, zer_nh)
    hs, hd, cmat = pl.pallas_call(
        _tc_post_body,
        out_shape=[jax.ShapeDtypeStruct((N, H), f32),
                   jax.ShapeDtypeStruct((N, H), f32),
                   jax.ShapeDtypeStruct((DE, H), f32)],
    )(parts2[0], parts2[1], dinv, Wg1, bg1.reshape(1, H), Wd1,
      We, be.reshape(1, H), bd1.reshape(1, H))

    g2 = _sc_decoder(hs, hd, src2, dst2)

    out = pl.pallas_call(
        _tc_final_body,
        grid=(E // _EB,),
        in_specs=[
            pl.BlockSpec((_EB, H), lambda i: (i, 0)),
            pl.BlockSpec((DE, _EB), lambda i: (0, i)),
            pl.BlockSpec((DE, H), lambda i: (0, 0)),
            pl.BlockSpec((H, 1), lambda i: (0, 0)),
            pl.BlockSpec((1, 1), lambda i: (0, 0)),
        ],
        out_specs=pl.BlockSpec((1, _EB // H, H), lambda i: (i, 0, 0)),
        out_shape=jax.ShapeDtypeStruct((E // _EB, _EB // H, H), f32),
    )(g2, edge_attr.T, cmat, Wd2, bd2.reshape(1, 1))
    return out.reshape(E, 1)


# speedup vs baseline: 1.7605x; 1.0191x over previous
import functools

import jax
import jax.numpy as jnp
from jax import lax
from jax.experimental import pallas as pl
from jax.experimental.pallas import tpu as pltpu
from jax.experimental.pallas import tpu_sc as plsc

N = 10000
E = 320000
H = 128
DE = 16
NC, NS = 2, 16
NW = NC * NS
K = 80
CPW = (E // NW) // K
ROWS = E // K

_mesh = plsc.VectorSubcoreMesh(core_axis_name="c", subcore_axis_name="s")


@functools.partial(
    pl.kernel, mesh=_mesh,
    out_type=jax.ShapeDtypeStruct((NC, N), jnp.float32),
    scratch_types=[
        pltpu.VMEM((CPW, K), jnp.int32),
        pltpu.VMEM((CPW, K), jnp.float32),
        pltpu.VMEM_SHARED((N,), jnp.float32),
    ],
)
def _sc_deg(dst_hbm, ew_hbm, zer_hbm, out_hbm, didx, ewv, acc):
    c = lax.axis_index("c")
    s = lax.axis_index("s")
    w = c * NS + s
    pltpu.sync_copy(dst_hbm.at[w], didx)
    pltpu.sync_copy(ew_hbm.at[w], ewv)

    @pl.when(s == 0)
    def _():
        pltpu.sync_copy(zer_hbm, acc)

    plsc.subcore_barrier()

    def chunk(j, carry):
        pltpu.sync_copy(ewv.at[j], acc.at[didx.at[j]], add=True)
        return carry

    lax.fori_loop(0, CPW, chunk, 0)
    plsc.subcore_barrier()

    @pl.when(s == 0)
    def _():
        pltpu.sync_copy(acc, out_hbm.at[c])


@functools.partial(
    pl.kernel, mesh=_mesh,
    out_type=jax.ShapeDtypeStruct((NC, N, H), jnp.float32),
    scratch_types=[
        pltpu.VMEM((8, K), jnp.int32),
        pltpu.VMEM((2, 1, K), jnp.float32),
        pltpu.VMEM((2, K, H), jnp.float32),
        pltpu.VMEM_SHARED((N, H), jnp.float32),
        pltpu.SemaphoreType.DMA,
        pltpu.SemaphoreType.DMA,
        pltpu.SemaphoreType.DMA,
        pltpu.SemaphoreType.DMA,
        pltpu.SemaphoreType.DMA,
        pltpu.SemaphoreType.DMA,
    ],
)
def _sc_layer(hp_hbm, sde_hbm, ew_hbm, zer_hbm, out_hbm,
              idx4, ewb, rows, acc, gsem0, gsem1, isem0, isem1, isem2, isem3):
    c = lax.axis_index("c")
    s = lax.axis_index("s")
    w = c * NS + s
    gsems = (gsem0, gsem1)
    isems = (isem0, isem1, isem2, isem3)

    @pl.when(s == 0)
    def _():
        pltpu.sync_copy(zer_hbm, acc)

    plsc.subcore_barrier()

    def issue_idx(cur, r):
        pltpu.async_copy(sde_hbm.at[w, cur], idx4.at[pl.ds(2 * r, 2)],
                         isems[r])

    def wait_idx(r):
        pltpu.make_async_copy(sde_hbm.at[w, 0], idx4.at[pl.ds(2 * r, 2)],
                              isems[r]).wait()

    def issue_gather(cur, r, b):
        pltpu.async_copy(hp_hbm.at[idx4.at[2 * r]], rows.at[b], gsems[b])
        pltpu.async_copy(ew_hbm.at[w, cur], ewb.at[b], gsems[b])

    def wait_gather(r, b):
        pltpu.make_async_copy(hp_hbm.at[idx4.at[2 * r]], rows.at[b],
                              gsems[b]).wait()
        pltpu.make_async_copy(ew_hbm.at[w, 0], ewb.at[b], gsems[b]).wait()

    for t in range(4):
        issue_idx(t, t)
    wait_idx(0)
    wait_idx(1)
    issue_gather(0, 0, 0)
    issue_gather(1, 1, 1)

    def process(cur, t):
        b = t % 2
        r2 = (t + 2) % 4
        wait_gather(t, b)

        def scale16(g, c2):
            wv = ewb[b, 0, pl.ds(g * 16, 16)]
            for l in range(16):
                wgt = wv[l]
                i = g * 16 + l
                for q in range(H // 16):
                    sl = pl.ds(q * 16, 16)
                    rows[b, i, sl] = rows[b, i, sl] * wgt
            return c2

        lax.fori_loop(0, K // 16, scale16, 0)
        pltpu.sync_copy(rows.at[b], acc.at[idx4.at[2 * t + 1]], add=True)

        def prefetch_idx():
            issue_idx(cur + 4, t)

        def next_gather():
            wait_idx(r2)
            issue_gather(cur + 2, r2, b)

        if isinstance(cur, int):
            if cur + 4 < CPW:
                prefetch_idx()
            if cur + 2 < CPW:
                next_gather()
        else:
            @pl.when(cur + 4 < CPW)
            def _():
                prefetch_idx()

            @pl.when(cur + 2 < CPW)
            def _():
                next_gather()

    @pl.loop(0, CPW - 1, step=4)
    def _(j):
        for t in range(4):
            process(j + t, t)

    process(CPW - 1, 0)
    plsc.subcore_barrier()

    @pl.when(s == 0)
    def _():
        pltpu.sync_copy(acc, out_hbm.at[c])


@functools.partial(
    pl.kernel, mesh=_mesh,
    out_type=jax.ShapeDtypeStruct((E, H), jnp.float32),
    scratch_types=[
        pltpu.VMEM((CPW, K), jnp.int32),
        pltpu.VMEM((CPW, K), jnp.int32),
        pltpu.VMEM((2, K, H), jnp.float32),
        pltpu.VMEM((2, K, H), jnp.float32),
        pltpu.VMEM((2, K, H), jnp.float32),
        pltpu.SemaphoreType.DMA,
        pltpu.SemaphoreType.DMA,
        pltpu.SemaphoreType.DMA,
        pltpu.SemaphoreType.DMA,
    ],
)
def _sc_decoder(hs_hbm, hd_hbm, src_hbm, dst_hbm, out_hbm,
                sidx_all, didx_all, ra, rb, ob, gsem0, gsem1, osem0, osem1):
    c = lax.axis_index("c")
    s = lax.axis_index("s")
    w = c * NS + s
    base = w * CPW
    pltpu.sync_copy(src_hbm.at[w], sidx_all)
    pltpu.sync_copy(dst_hbm.at[w], didx_all)
    gsems = (gsem0, gsem1)
    osems = (osem0, osem1)

    def issue(cur, b):
        pltpu.async_copy(hs_hbm.at[sidx_all.at[cur]], ra.at[b], gsems[b])
        pltpu.async_copy(hd_hbm.at[didx_all.at[cur]], rb.at[b], gsems[b])

    def wait_gather(cur, b):
        pltpu.make_async_copy(hs_hbm.at[sidx_all.at[cur]], ra.at[b],
                              gsems[b]).wait()
        pltpu.make_async_copy(hd_hbm.at[didx_all.at[cur]], rb.at[b],
                              gsems[b]).wait()

    def wait_store(cur, b):
        pltpu.make_async_copy(ob.at[b], out_hbm.at[pl.ds((base + cur) * K, K)],
                              osems[b]).wait()

    issue(0, 0)
    issue(1, 1)

    def process(cur, b):
        if isinstance(cur, int):
            if cur >= 2:
                wait_store(cur - 2, b)
        else:
            @pl.when(cur >= 2)
            def _():
                wait_store(cur - 2, b)
        wait_gather(cur, b)

        def addrow(i, c2):
            for q in range(H // 16):
                sl = pl.ds(q * 16, 16)
                ob[b, i, sl] = ra[b, i, sl] + rb[b, i, sl]
            return c2

        lax.fori_loop(0, K, addrow, 0)
        pltpu.async_copy(ob.at[b], out_hbm.at[pl.ds((base + cur) * K, K)],
                         osems[b])
        if isinstance(cur, int):
            if cur + 2 < CPW:
                issue(cur + 2, b)
        else:
            @pl.when(cur + 2 < CPW)
            def _():
                issue(cur + 2, b)

    @pl.loop(0, CPW - 1, step=2)
    def _(j):
        process(j, 0)
        process(j + 1, 1)

    process(CPW - 1, 0)
    wait_store(CPW - 2, 1)
    wait_store(CPW - 1, 0)


def _tc_prep_body(x_ref, wx_ref, bx_ref, dpt_ref, h1p_ref, dinv_ref):
    dp = dpt_ref[...]
    deg = dp[:, 0:1] + dp[:, 1:2]
    dinv = jnp.where(deg > 0, lax.rsqrt(jnp.maximum(deg, 1e-12)), 0.0)
    h = jnp.dot(x_ref[...], wx_ref[...],
                preferred_element_type=jnp.float32) + bx_ref[...]
    h1p_ref[...] = h * dinv
    dinv_ref[...] = dinv


def _tc_mid_body(p_ref, dinv_ref, w_ref, b_ref, out_ref):
    dinv = dinv_ref[...]
    agg = (p_ref[0] + p_ref[1]) * dinv
    h = jnp.dot(agg, w_ref[...], preferred_element_type=jnp.float32) + b_ref[...]
    out_ref[...] = jnp.maximum(h, 0.0) * dinv


def _tc_post_body(p_ref, dinv_ref, w_ref, b_ref, wd1_ref, we_ref,
                  be_ref, bd1_ref, hs_ref, hd_ref, cmat_ref):
    agg = (p_ref[0] + p_ref[1]) * dinv_ref[...]
    h = jnp.dot(agg, w_ref[...], preferred_element_type=jnp.float32) + b_ref[...]
    h = jnp.maximum(h, 0.0)
    wd1 = wd1_ref[...]
    a = wd1[0:H, :]
    b2 = wd1[H:2 * H, :]
    c0 = wd1[2 * H:3 * H, :]
    cmat = jnp.dot(we_ref[...], c0, preferred_element_type=jnp.float32)
    bconst = bd1_ref[...] + jnp.dot(be_ref[...], c0,
                                    preferred_element_type=jnp.float32)
    hs_ref[...] = jnp.dot(h, a, preferred_element_type=jnp.float32)
    hd_ref[...] = jnp.dot(h, b2, preferred_element_type=jnp.float32) + bconst
    cmat_ref[...] = cmat


def _tc_final_body(g_ref, eat_ref, cmat_ref, wd2_ref, bd2_ref, out_ref):
    v = g_ref[...] + lax.dot_general(eat_ref[...], cmat_ref[...],
                                     (((0,), (0,)), ((), ())),
                                     preferred_element_type=jnp.float32)
    v = jnp.maximum(v, 0.0)
    v3 = v.reshape(_EB // H, H, H)
    r = lax.dot_general(v3, wd2_ref[...], (((2,), (0,)), ((), ())),
                        preferred_element_type=jnp.float32)
    out_ref[...] = (r[:, :, 0] + bd2_ref[...])[None]


_EB = 3200


def kernel(x, edge_index, edge_attr, edge_weight,
           Wx, bx, We, be, Wg0, bg0, Wg1, bg1, Wd1, bd1, Wd2, bd2):
    f32 = jnp.float32
    src2 = edge_index[0].reshape(NW, CPW, K)
    dst2 = edge_index[1].reshape(NW, CPW, K)
    ew2 = edge_weight.reshape(NW, CPW, K)
    zer_n = jnp.zeros((N,), f32)
    zer_nh = jnp.zeros((N, H), f32)

    sde = jnp.stack([src2, dst2], axis=2)
    ew4 = ew2.reshape(NW, CPW, 1, K)

    deg_parts = _sc_deg(dst2, ew2, zer_n)
    dpt = deg_parts.T

    h1p, dinv = pl.pallas_call(
        _tc_prep_body,
        out_shape=[jax.ShapeDtypeStruct((N, H), f32),
                   jax.ShapeDtypeStruct((N, 1), f32)],
    )(x, Wx, bx.reshape(1, H), dpt)

    parts1 = _sc_layer(h1p, sde, ew4, zer_nh)
    h2p = pl.pallas_call(
        _tc_mid_body,
        out_shape=jax.ShapeDtypeStruct((N, H), f32),
    )(parts1, dinv, Wg0, bg0.reshape(1, H))

    parts2 = _sc_layer(h2p, sde, ew4, zer_nh)
    hs, hd, cmat = pl.pallas_call(
        _tc_post_body,
        out_shape=[jax.ShapeDtypeStruct((N, H), f32),
                   jax.ShapeDtypeStruct((N, H), f32),
                   jax.ShapeDtypeStruct((DE, H), f32)],
    )(parts2, dinv, Wg1, bg1.reshape(1, H), Wd1,
      We, be.reshape(1, H), bd1.reshape(1, H))

    g2 = _sc_decoder(hs, hd, src2, dst2)

    out = pl.pallas_call(
        _tc_final_body,
        grid=(E // _EB,),
        in_specs=[
            pl.BlockSpec((_EB, H), lambda i: (i, 0)),
            pl.BlockSpec((DE, _EB), lambda i: (0, i)),
            pl.BlockSpec((DE, H), lambda i: (0, 0)),
            pl.BlockSpec((H, 1), lambda i: (0, 0)),
            pl.BlockSpec((1, 1), lambda i: (0, 0)),
        ],
        out_specs=pl.BlockSpec((1, _EB // H, H), lambda i: (i, 0, 0)),
        out_shape=jax.ShapeDtypeStruct((E // _EB, _EB // H, H), f32),
    )(g2, edge_attr.T, cmat, Wd2, bd2.reshape(1, 1))
    return out.reshape(E, 1)


# speedup vs baseline: 1.8070x; 1.0264x over previous
import functools

import jax
import jax.numpy as jnp
from jax import lax
from jax.experimental import pallas as pl
from jax.experimental.pallas import tpu as pltpu
from jax.experimental.pallas import tpu_sc as plsc

N = 10000
E = 320000
H = 128
DE = 16
NC, NS = 2, 16
NW = NC * NS
K = 80
CPW = (E // NW) // K
ROWS = E // K

_mesh = plsc.VectorSubcoreMesh(core_axis_name="c", subcore_axis_name="s")


@functools.partial(
    pl.kernel, mesh=_mesh,
    out_type=jax.ShapeDtypeStruct((NC, N), jnp.float32),
    scratch_types=[
        pltpu.VMEM((CPW, K), jnp.int32),
        pltpu.VMEM((CPW, K), jnp.float32),
        pltpu.VMEM_SHARED((N,), jnp.float32),
    ],
)
def _sc_deg(ei_hbm, ew_hbm, zer_hbm, out_hbm, didx, ewv, acc):
    c = lax.axis_index("c")
    s = lax.axis_index("s")
    w = c * NS + s
    pltpu.sync_copy(ei_hbm.at[1, w], didx)
    pltpu.sync_copy(ew_hbm.at[w], ewv)

    @pl.when(s == 0)
    def _():
        pltpu.sync_copy(zer_hbm, acc)

    plsc.subcore_barrier()

    def chunk(j, carry):
        pltpu.sync_copy(ewv.at[j], acc.at[didx.at[j]], add=True)
        return carry

    lax.fori_loop(0, CPW, chunk, 0)
    plsc.subcore_barrier()

    @pl.when(s == 0)
    def _():
        pltpu.sync_copy(acc, out_hbm.at[c])


@functools.partial(
    pl.kernel, mesh=_mesh,
    out_type=jax.ShapeDtypeStruct((NC, N, H), jnp.float32),
    scratch_types=[
        pltpu.VMEM((8, 1, K), jnp.int32),
        pltpu.VMEM((2, 1, K), jnp.float32),
        pltpu.VMEM((2, K, H), jnp.float32),
        pltpu.VMEM_SHARED((N, H), jnp.float32),
        pltpu.SemaphoreType.DMA,
        pltpu.SemaphoreType.DMA,
        pltpu.SemaphoreType.DMA,
        pltpu.SemaphoreType.DMA,
        pltpu.SemaphoreType.DMA,
        pltpu.SemaphoreType.DMA,
    ],
)
def _sc_layer(hp_hbm, ei_hbm, ew_hbm, zer_hbm, out_hbm,
              idx4, ewb, rows, acc, gsem0, gsem1, isem0, isem1, isem2, isem3):
    c = lax.axis_index("c")
    s = lax.axis_index("s")
    w = c * NS + s
    gsems = (gsem0, gsem1)
    isems = (isem0, isem1, isem2, isem3)

    @pl.when(s == 0)
    def _():
        pltpu.sync_copy(zer_hbm, acc)

    plsc.subcore_barrier()

    def issue_idx(cur, r):
        pltpu.async_copy(ei_hbm.at[0, w, cur], idx4.at[2 * r], isems[r])
        pltpu.async_copy(ei_hbm.at[1, w, cur], idx4.at[2 * r + 1], isems[r])

    def wait_idx(r):
        pltpu.make_async_copy(ei_hbm.at[0, w, 0], idx4.at[2 * r],
                              isems[r]).wait()
        pltpu.make_async_copy(ei_hbm.at[1, w, 0], idx4.at[2 * r + 1],
                              isems[r]).wait()

    def issue_gather(cur, r, b):
        pltpu.async_copy(hp_hbm.at[idx4.at[2 * r, 0]], rows.at[b], gsems[b])
        pltpu.async_copy(ew_hbm.at[w, cur], ewb.at[b], gsems[b])

    def wait_gather(r, b):
        pltpu.make_async_copy(hp_hbm.at[idx4.at[2 * r, 0]], rows.at[b],
                              gsems[b]).wait()
        pltpu.make_async_copy(ew_hbm.at[w, 0], ewb.at[b], gsems[b]).wait()

    for t in range(4):
        issue_idx(t, t)
    wait_idx(0)
    wait_idx(1)
    issue_gather(0, 0, 0)
    issue_gather(1, 1, 1)

    def process(cur, t):
        b = t % 2
        r2 = (t + 2) % 4
        wait_gather(t, b)

        def scale16(g, c2):
            wv = ewb[b, 0, pl.ds(g * 16, 16)]
            for l in range(16):
                wgt = wv[l]
                i = g * 16 + l
                for q in range(H // 16):
                    sl = pl.ds(q * 16, 16)
                    rows[b, i, sl] = rows[b, i, sl] * wgt
            return c2

        lax.fori_loop(0, K // 16, scale16, 0)
        pltpu.sync_copy(rows.at[b], acc.at[idx4.at[2 * t + 1, 0]], add=True)

        def prefetch_idx():
            issue_idx(cur + 4, t)

        def next_gather():
            wait_idx(r2)
            issue_gather(cur + 2, r2, b)

        if isinstance(cur, int):
            if cur + 4 < CPW:
                prefetch_idx()
            if cur + 2 < CPW:
                next_gather()
        else:
            @pl.when(cur + 4 < CPW)
            def _():
                prefetch_idx()

            @pl.when(cur + 2 < CPW)
            def _():
                next_gather()

    @pl.loop(0, CPW - 1, step=4)
    def _(j):
        for t in range(4):
            process(j + t, t)

    process(CPW - 1, 0)
    plsc.subcore_barrier()

    @pl.when(s == 0)
    def _():
        pltpu.sync_copy(acc, out_hbm.at[c])


@functools.partial(
    pl.kernel, mesh=_mesh,
    out_type=jax.ShapeDtypeStruct((E, H), jnp.float32),
    scratch_types=[
        pltpu.VMEM((CPW, K), jnp.int32),
        pltpu.VMEM((CPW, K), jnp.int32),
        pltpu.VMEM((2, K, H), jnp.float32),
        pltpu.VMEM((2, K, H), jnp.float32),
        pltpu.VMEM((2, K, H), jnp.float32),
        pltpu.SemaphoreType.DMA,
        pltpu.SemaphoreType.DMA,
        pltpu.SemaphoreType.DMA,
        pltpu.SemaphoreType.DMA,
    ],
)
def _sc_decoder(hs_hbm, hd_hbm, ei_hbm, out_hbm,
                sidx_all, didx_all, ra, rb, ob, gsem0, gsem1, osem0, osem1):
    c = lax.axis_index("c")
    s = lax.axis_index("s")
    w = c * NS + s
    base = w * CPW
    pltpu.sync_copy(ei_hbm.at[0, w], sidx_all)
    pltpu.sync_copy(ei_hbm.at[1, w], didx_all)
    gsems = (gsem0, gsem1)
    osems = (osem0, osem1)

    def issue(cur, b):
        pltpu.async_copy(hs_hbm.at[sidx_all.at[cur]], ra.at[b], gsems[b])
        pltpu.async_copy(hd_hbm.at[didx_all.at[cur]], rb.at[b], gsems[b])

    def wait_gather(cur, b):
        pltpu.make_async_copy(hs_hbm.at[sidx_all.at[cur]], ra.at[b],
                              gsems[b]).wait()
        pltpu.make_async_copy(hd_hbm.at[didx_all.at[cur]], rb.at[b],
                              gsems[b]).wait()

    def wait_store(cur, b):
        pltpu.make_async_copy(ob.at[b], out_hbm.at[pl.ds((base + cur) * K, K)],
                              osems[b]).wait()

    issue(0, 0)
    issue(1, 1)

    def process(cur, b):
        if isinstance(cur, int):
            if cur >= 2:
                wait_store(cur - 2, b)
        else:
            @pl.when(cur >= 2)
            def _():
                wait_store(cur - 2, b)
        wait_gather(cur, b)

        def addrow(i, c2):
            for q in range(H // 16):
                sl = pl.ds(q * 16, 16)
                ob[b, i, sl] = ra[b, i, sl] + rb[b, i, sl]
            return c2

        lax.fori_loop(0, K, addrow, 0)
        pltpu.async_copy(ob.at[b], out_hbm.at[pl.ds((base + cur) * K, K)],
                         osems[b])
        if isinstance(cur, int):
            if cur + 2 < CPW:
                issue(cur + 2, b)
        else:
            @pl.when(cur + 2 < CPW)
            def _():
                issue(cur + 2, b)

    @pl.loop(0, CPW - 1, step=2)
    def _(j):
        process(j, 0)
        process(j + 1, 1)

    process(CPW - 1, 0)
    wait_store(CPW - 2, 1)
    wait_store(CPW - 1, 0)


def _tc_prep_body(x_ref, wx_ref, bx_ref, dpt_ref, h1p_ref, dinv_ref):
    dp = dpt_ref[...]
    deg = dp[:, 0:1] + dp[:, 1:2]
    dinv = jnp.where(deg > 0, lax.rsqrt(jnp.maximum(deg, 1e-12)), 0.0)
    h = jnp.dot(x_ref[...], wx_ref[...],
                preferred_element_type=jnp.float32) + bx_ref[...]
    h1p_ref[...] = h * dinv
    dinv_ref[...] = dinv


def _tc_mid_body(p_ref, dinv_ref, w_ref, b_ref, out_ref):
    dinv = dinv_ref[...]
    agg = (p_ref[0] + p_ref[1]) * dinv
    h = jnp.dot(agg, w_ref[...], preferred_element_type=jnp.float32) + b_ref[...]
    out_ref[...] = jnp.maximum(h, 0.0) * dinv


def _tc_post_body(p_ref, dinv_ref, w_ref, b_ref, wd1_ref, we_ref,
                  be_ref, bd1_ref, hs_ref, hd_ref, cmat_ref):
    agg = (p_ref[0] + p_ref[1]) * dinv_ref[...]
    h = jnp.dot(agg, w_ref[...], preferred_element_type=jnp.float32) + b_ref[...]
    h = jnp.maximum(h, 0.0)
    wd1 = wd1_ref[...]
    a = wd1[0:H, :]
    b2 = wd1[H:2 * H, :]
    c0 = wd1[2 * H:3 * H, :]
    cmat = jnp.dot(we_ref[...], c0, preferred_element_type=jnp.float32)
    bconst = bd1_ref[...] + jnp.dot(be_ref[...], c0,
                                    preferred_element_type=jnp.float32)
    hs_ref[...] = jnp.dot(h, a, preferred_element_type=jnp.float32)
    hd_ref[...] = jnp.dot(h, b2, preferred_element_type=jnp.float32) + bconst
    cmat_ref[...] = cmat


def _tc_final_body(g_ref, eat_ref, cmat_ref, wd2_ref, bd2_ref, out_ref):
    v = g_ref[...] + lax.dot_general(eat_ref[...], cmat_ref[...],
                                     (((0,), (0,)), ((), ())),
                                     preferred_element_type=jnp.float32)
    v = jnp.maximum(v, 0.0)
    v3 = v.reshape(_EB // H, H, H)
    r = lax.dot_general(v3, wd2_ref[...], (((2,), (0,)), ((), ())),
                        preferred_element_type=jnp.float32)
    out_ref[...] = (r[:, :, 0] + bd2_ref[...])[None]


_EB = 3200


def kernel(x, edge_index, edge_attr, edge_weight,
           Wx, bx, We, be, Wg0, bg0, Wg1, bg1, Wd1, bd1, Wd2, bd2):
    f32 = jnp.float32
    ei4 = edge_index.reshape(2, NW, CPW, K)
    ei5 = edge_index.reshape(2, NW, CPW, 1, K)
    ew2 = edge_weight.reshape(NW, CPW, K)
    zer_n = jnp.zeros((N,), f32)
    zer_nh = jnp.zeros((N, H), f32)

    ew4 = edge_weight.reshape(NW, CPW, 1, K)

    deg_parts = _sc_deg(ei4, ew2, zer_n)
    dpt = deg_parts.T

    h1p, dinv = pl.pallas_call(
        _tc_prep_body,
        out_shape=[jax.ShapeDtypeStruct((N, H), f32),
                   jax.ShapeDtypeStruct((N, 1), f32)],
    )(x, Wx, bx.reshape(1, H), dpt)

    parts1 = _sc_layer(h1p, ei5, ew4, zer_nh)
    h2p = pl.pallas_call(
        _tc_mid_body,
        out_shape=jax.ShapeDtypeStruct((N, H), f32),
    )(parts1, dinv, Wg0, bg0.reshape(1, H))

    parts2 = _sc_layer(h2p, ei5, ew4, zer_nh)
    hs, hd, cmat = pl.pallas_call(
        _tc_post_body,
        out_shape=[jax.ShapeDtypeStruct((N, H), f32),
                   jax.ShapeDtypeStruct((N, H), f32),
                   jax.ShapeDtypeStruct((DE, H), f32)],
    )(parts2, dinv, Wg1, bg1.reshape(1, H), Wd1,
      We, be.reshape(1, H), bd1.reshape(1, H))

    g2 = _sc_decoder(hs, hd, ei4)

    out = pl.pallas_call(
        _tc_final_body,
        grid=(E // _EB,),
        in_specs=[
            pl.BlockSpec((_EB, H), lambda i: (i, 0)),
            pl.BlockSpec((DE, _EB), lambda i: (0, i)),
            pl.BlockSpec((DE, H), lambda i: (0, 0)),
            pl.BlockSpec((H, 1), lambda i: (0, 0)),
            pl.BlockSpec((1, 1), lambda i: (0, 0)),
        ],
        out_specs=pl.BlockSpec((1, _EB // H, H), lambda i: (i, 0, 0)),
        out_shape=jax.ShapeDtypeStruct((E // _EB, _EB // H, H), f32),
    )(g2, edge_attr.T, cmat, Wd2, bd2.reshape(1, 1))
    return out.reshape(E, 1)
